# baseline (speedup 1.0000x reference)
import functools

import jax
import jax.numpy as jnp
from jax import lax
from jax.experimental import pallas as pl
from jax.experimental.pallas import tpu as pltpu
from jax.experimental.pallas import tpu_sc as plsc

_B = 32
_N = 65536
_RES = 4
_V = _RES ** 3
_C = 40
_CPAD = 48
_L = 16

_CH = 4096
_NCHUNK = _N // _CH
_GROUPS = _CH // _L


def _sc_body(x_hbm, w_hbm, b_hbm, out_hbm,
             buf0, buf1, hist, counts, wv, bv, ov, sem0, sem1):
    ncores = 2
    wid = lax.axis_index("s") * ncores + lax.axis_index("c")

    iota = lax.iota(jnp.int32, _L)
    ones = jnp.ones((_L,), jnp.float32)
    zeros = jnp.zeros((_L,), jnp.float32)
    c0 = jnp.zeros((_L,), jnp.int32)
    c1 = jnp.full((_L,), 1, jnp.int32)
    c2 = jnp.full((_L,), 2, jnp.int32)

    pltpu.sync_copy(w_hbm, wv)
    pltpu.sync_copy(b_hbm, bv)

    for j in range(_V):
        hist[pl.ds(j * _L, _L)] = zeros

    def process(buf):
        @plsc.parallel_loop(0, _GROUPS, unroll=8)
        def body(g):
            pidx = iota + g * _L
            xg = plsc.load_gather(buf, [pidx, c0])
            yg = plsc.load_gather(buf, [pidx, c1])
            zg = plsc.load_gather(buf, [pidx, c2])
            dx = (xg * 4.0).astype(jnp.int32)
            dy = (yg * 4.0).astype(jnp.int32)
            dz = (zg * 4.0).astype(jnp.int32)
            sidx = dx * 256 + dy * 64 + dz * 16 + iota
            plsc.addupdate_scatter(hist, [sidx], ones)

    bufs = (buf0, buf1)
    sems = (sem0, sem1)
    cur = pltpu.async_copy(x_hbm.at[wid, pl.ds(0, _CH), :], buf0, sem0)
    for c in range(_NCHUNK):
        nxt = None
        if c + 1 < _NCHUNK:
            nxt = pltpu.async_copy(
                x_hbm.at[wid, pl.ds((c + 1) * _CH, _CH), :],
                bufs[(c + 1) % 2], sems[(c + 1) % 2])
        cur.wait()
        process(bufs[c % 2])
        cur = nxt

    for j in range(_V // _L):
        acc = zeros
        for l in range(_L):
            acc = acc + plsc.load_gather(hist, [iota * _L + (j * _L * _L + l)])
        counts[pl.ds(j * _L, _L)] = acc

    acc0 = zeros
    acc1 = zeros
    acc2 = zeros
    for v in range(_V):
        cv = plsc.load_gather(counts, [jnp.full((_L,), v, jnp.int32)])
        acc0 = acc0 + cv * wv[pl.ds(v * _CPAD, _L)]
        acc1 = acc1 + cv * wv[pl.ds(v * _CPAD + _L, _L)]
        acc2 = acc2 + cv * wv[pl.ds(v * _CPAD + 2 * _L, _L)]
    scale = jnp.float32(1.0 / _N)
    ov[pl.ds(0, _L)] = acc0 * scale + bv[pl.ds(0, _L)]
    ov[pl.ds(_L, _L)] = acc1 * scale + bv[pl.ds(_L, _L)]
    ov[pl.ds(2 * _L, _L)] = acc2 * scale + bv[pl.ds(2 * _L, _L)]
    pltpu.sync_copy(ov, out_hbm.at[wid])


@jax.jit
def _histogram_classify(x, wp, bp):
    mesh = plsc.VectorSubcoreMesh(core_axis_name="c", subcore_axis_name="s")
    fn = functools.partial(
        pl.kernel,
        mesh=mesh,
        compiler_params=pltpu.CompilerParams(
            needs_layout_passes=False, use_tc_tiling_on_sc=False),
        out_type=jax.ShapeDtypeStruct((_B, _CPAD), jnp.float32),
        scratch_types=[
            pltpu.VMEM((_CH, 3), jnp.float32),
            pltpu.VMEM((_CH, 3), jnp.float32),
            pltpu.VMEM((_L * _V,), jnp.float32),
            pltpu.VMEM((_V,), jnp.float32),
            pltpu.VMEM((_V * _CPAD,), jnp.float32),
            pltpu.VMEM((_CPAD,), jnp.float32),
            pltpu.VMEM((_CPAD,), jnp.float32),
            pltpu.SemaphoreType.DMA,
            pltpu.SemaphoreType.DMA,
        ],
    )(_sc_body)
    return fn(x, wp, bp)


def kernel(x, W, b):
    wp = jnp.zeros((_V, _CPAD), jnp.float32).at[:, :_C].set(W).reshape(-1)
    bp = jnp.zeros((_CPAD,), jnp.float32).at[:_C].set(b)
    out = _histogram_classify(x, wp, bp)
    return out[:, :_C]


# speedup vs baseline: 25.9347x; 25.9347x over previous
import functools

import jax
import jax.numpy as jnp
from jax import lax
from jax.experimental import pallas as pl
from jax.experimental.pallas import tpu as pltpu
from jax.experimental.pallas import tpu_sc as plsc

_B = 32
_N = 65536
_RES = 4
_V = _RES ** 3
_C = 40
_CPAD = 48
_L = 16

_CH = 4096
_NCHUNK = _N // _CH
_GROUPS = _CH // _L


def _sc_body(x_hbm, w_hbm, b_hbm, out_hbm,
             buf0, buf1, hist, counts, wv, bv, ov, sem0, sem1):
    ncores = 2
    wid = lax.axis_index("s") * ncores + lax.axis_index("c")

    iota = lax.iota(jnp.int32, _L)
    off0 = iota * 3
    ones = jnp.ones((_L,), jnp.float32)
    zeros = jnp.zeros((_L,), jnp.float32)

    pltpu.sync_copy(w_hbm, wv)
    pltpu.sync_copy(b_hbm, bv)

    for j in range(_V):
        hist[pl.ds(j * _L, _L)] = zeros

    def process(buf):
        @plsc.parallel_loop(0, _GROUPS, unroll=8)
        def body(g):
            offx = off0 + g * (_L * 3)
            xg = plsc.load_gather(buf, [offx])
            yg = plsc.load_gather(buf, [offx + 1])
            zg = plsc.load_gather(buf, [offx + 2])
            dx = (xg * 4.0).astype(jnp.int32)
            dy = (yg * 4.0).astype(jnp.int32)
            dz = (zg * 4.0).astype(jnp.int32)
            sidx = dx * 256 + dy * 64 + dz * 16 + iota
            plsc.addupdate_scatter(hist, [sidx], ones)

    bufs = (buf0, buf1)
    sems = (sem0, sem1)
    cur = pltpu.async_copy(x_hbm.at[wid, pl.ds(0, _CH * 3)], buf0, sem0)
    for c in range(_NCHUNK):
        nxt = None
        if c + 1 < _NCHUNK:
            nxt = pltpu.async_copy(
                x_hbm.at[wid, pl.ds((c + 1) * _CH * 3, _CH * 3)],
                bufs[(c + 1) % 2], sems[(c + 1) % 2])
        cur.wait()
        process(bufs[c % 2])
        cur = nxt

    for j in range(_V // _L):
        acc = zeros
        for l in range(_L):
            acc = acc + plsc.load_gather(hist, [iota * _L + (j * _L * _L + l)])
        counts[pl.ds(j * _L, _L)] = acc

    acc0 = zeros
    acc1 = zeros
    acc2 = zeros
    for v in range(_V):
        cv = plsc.load_gather(counts, [jnp.full((_L,), v, jnp.int32)])
        acc0 = acc0 + cv * wv[pl.ds(v * _CPAD, _L)]
        acc1 = acc1 + cv * wv[pl.ds(v * _CPAD + _L, _L)]
        acc2 = acc2 + cv * wv[pl.ds(v * _CPAD + 2 * _L, _L)]
    scale = jnp.float32(1.0 / _N)
    ov[pl.ds(0, _L)] = acc0 * scale + bv[pl.ds(0, _L)]
    ov[pl.ds(_L, _L)] = acc1 * scale + bv[pl.ds(_L, _L)]
    ov[pl.ds(2 * _L, _L)] = acc2 * scale + bv[pl.ds(2 * _L, _L)]
    pltpu.sync_copy(ov, out_hbm.at[wid])


@jax.jit
def _histogram_classify(x, wp, bp):
    mesh = plsc.VectorSubcoreMesh(core_axis_name="c", subcore_axis_name="s")
    fn = functools.partial(
        pl.kernel,
        mesh=mesh,
        compiler_params=pltpu.CompilerParams(
            needs_layout_passes=False, use_tc_tiling_on_sc=False),
        out_type=jax.ShapeDtypeStruct((_B, _CPAD), jnp.float32),
        scratch_types=[
            pltpu.VMEM((_CH * 3,), jnp.float32),
            pltpu.VMEM((_CH * 3,), jnp.float32),
            pltpu.VMEM((_L * _V,), jnp.float32),
            pltpu.VMEM((_V,), jnp.float32),
            pltpu.VMEM((_V * _CPAD,), jnp.float32),
            pltpu.VMEM((_CPAD,), jnp.float32),
            pltpu.VMEM((_CPAD,), jnp.float32),
            pltpu.SemaphoreType.DMA,
            pltpu.SemaphoreType.DMA,
        ],
    )(_sc_body)
    return fn(x, wp, bp)


def kernel(x, W, b):
    wp = jnp.zeros((_V, _CPAD), jnp.float32).at[:, :_C].set(W).reshape(-1)
    bp = jnp.zeros((_CPAD,), jnp.float32).at[:_C].set(b)
    out = _histogram_classify(x.reshape(_B, _N * 3), wp, bp)
    return out[:, :_C]


# speedup vs baseline: 108.8813x; 4.1983x over previous
import functools

import jax
import jax.numpy as jnp
from jax import lax
from jax.experimental import pallas as pl
from jax.experimental.pallas import tpu as pltpu
from jax.experimental.pallas import tpu_sc as plsc

_B = 32
_N = 65536
_RES = 4
_V = _RES ** 3
_C = 40
_CPAD = 48
_L = 16

_CH = 4096
_NCHUNK = _N // _CH
_GROUPS = _CH // _L


def _sc_body(x_hbm, w_hbm, b_hbm, out_hbm,
             bx0, by0, bz0, bx1, by1, bz1,
             hist, counts, wv, bv, ov, sem0, sem1):
    ncores = 2
    wid = lax.axis_index("s") * ncores + lax.axis_index("c")

    iota = lax.iota(jnp.int32, _L)
    ones = jnp.ones((_L,), jnp.float32)
    zeros = jnp.zeros((_L,), jnp.float32)

    pltpu.sync_copy(w_hbm, wv)
    pltpu.sync_copy(b_hbm, bv)

    for j in range(_V):
        hist[pl.ds(j * _L, _L)] = zeros

    def process(bx, by, bz):
        @plsc.parallel_loop(0, _GROUPS, unroll=8)
        def body(g):
            o = pl.ds(g * _L, _L)
            xg = bx[o]
            yg = by[o]
            zg = bz[o]
            dx = (xg * 4.0).astype(jnp.int32)
            dy = (yg * 4.0).astype(jnp.int32)
            dz = (zg * 4.0).astype(jnp.int32)
            sidx = dx * 256 + dy * 64 + dz * 16 + iota
            plsc.addupdate_scatter(hist, [sidx], ones)

    def start(c, slot):
        sem = sems[slot]
        src = pl.ds(c * _CH, _CH)
        return (pltpu.async_copy(x_hbm.at[wid, src], bufs[slot][0], sem),
                pltpu.async_copy(x_hbm.at[_B + wid, src], bufs[slot][1], sem),
                pltpu.async_copy(x_hbm.at[2 * _B + wid, src], bufs[slot][2], sem))

    bufs = ((bx0, by0, bz0), (bx1, by1, bz1))
    sems = (sem0, sem1)
    cur = start(0, 0)
    for c in range(_NCHUNK):
        nxt = None
        if c + 1 < _NCHUNK:
            nxt = start(c + 1, (c + 1) % 2)
        for h in cur:
            h.wait()
        process(*bufs[c % 2])
        cur = nxt

    for j in range(_V // _L):
        acc = zeros
        for l in range(_L):
            acc = acc + plsc.load_gather(hist, [iota * _L + (j * _L * _L + l)])
        counts[pl.ds(j * _L, _L)] = acc

    acc0 = zeros
    acc1 = zeros
    acc2 = zeros
    for v in range(_V):
        cv = plsc.load_gather(counts, [jnp.full((_L,), v, jnp.int32)])
        acc0 = acc0 + cv * wv[pl.ds(v * _CPAD, _L)]
        acc1 = acc1 + cv * wv[pl.ds(v * _CPAD + _L, _L)]
        acc2 = acc2 + cv * wv[pl.ds(v * _CPAD + 2 * _L, _L)]
    scale = jnp.float32(1.0 / _N)
    ov[pl.ds(0, _L)] = acc0 * scale + bv[pl.ds(0, _L)]
    ov[pl.ds(_L, _L)] = acc1 * scale + bv[pl.ds(_L, _L)]
    ov[pl.ds(2 * _L, _L)] = acc2 * scale + bv[pl.ds(2 * _L, _L)]
    pltpu.sync_copy(ov, out_hbm.at[wid])


@jax.jit
def _histogram_classify(xt, wp, bp):
    mesh = plsc.VectorSubcoreMesh(core_axis_name="c", subcore_axis_name="s")
    fn = functools.partial(
        pl.kernel,
        mesh=mesh,
        compiler_params=pltpu.CompilerParams(
            needs_layout_passes=False, use_tc_tiling_on_sc=False),
        out_type=jax.ShapeDtypeStruct((_B, _CPAD), jnp.float32),
        scratch_types=[
            pltpu.VMEM((_CH,), jnp.float32),
            pltpu.VMEM((_CH,), jnp.float32),
            pltpu.VMEM((_CH,), jnp.float32),
            pltpu.VMEM((_CH,), jnp.float32),
            pltpu.VMEM((_CH,), jnp.float32),
            pltpu.VMEM((_CH,), jnp.float32),
            pltpu.VMEM((_L * _V,), jnp.float32),
            pltpu.VMEM((_V,), jnp.float32),
            pltpu.VMEM((_V * _CPAD,), jnp.float32),
            pltpu.VMEM((_CPAD,), jnp.float32),
            pltpu.VMEM((_CPAD,), jnp.float32),
            pltpu.SemaphoreType.DMA,
            pltpu.SemaphoreType.DMA,
        ],
    )(_sc_body)
    return fn(xt, wp, bp)


def kernel(x, W, b):
    xt = jnp.transpose(x, (2, 0, 1)).reshape(3 * _B, _N)
    wp = jnp.zeros((_V, _CPAD), jnp.float32).at[:, :_C].set(W).reshape(-1)
    bp = jnp.zeros((_CPAD,), jnp.float32).at[:_C].set(b)
    out = _histogram_classify(xt, wp, bp)
    return out[:, :_C]


# speedup vs baseline: 163.3090x; 1.4999x over previous
import functools

import jax
import jax.numpy as jnp
from jax import lax
from jax.experimental import pallas as pl
from jax.experimental.pallas import tpu as pltpu
from jax.experimental.pallas import tpu_sc as plsc

_B = 32
_N = 65536
_RES = 4
_V = _RES ** 3
_C = 40
_CPAD = 48
_L = 16

_CH = 4096
_NCHUNK = _N // _CH
_GROUPS = _CH // _L


def _sc_body(x_hbm, w_hbm, b_hbm, out_hbm,
             bx0, by0, bz0, bx1, by1, bz1,
             hist, counts, wv, bv, ov, sem0, sem1):
    ncores = 2
    wid = lax.axis_index("s") * ncores + lax.axis_index("c")

    iota = lax.iota(jnp.int32, _L)
    ones = jnp.ones((_L,), jnp.float32)
    zeros = jnp.zeros((_L,), jnp.float32)

    pltpu.sync_copy(w_hbm, wv)
    pltpu.sync_copy(b_hbm, bv)

    for j in range(_V):
        hist[pl.ds(j * _L, _L)] = zeros

    def process(bx, by, bz):
        @plsc.parallel_loop(0, _GROUPS, unroll=8)
        def body(g):
            o = pl.ds(g * _L, _L)
            xg = bx[o]
            yg = by[o]
            zg = bz[o]
            dx = (xg * 4.0).astype(jnp.int32)
            dy = (yg * 4.0).astype(jnp.int32)
            dz = (zg * 4.0).astype(jnp.int32)
            sidx = dx * 256 + dy * 64 + dz * 16 + iota
            plsc.addupdate_scatter(hist, [sidx], ones)

    def start(c, slot):
        sem = sems[slot]
        src = pl.ds(c * _CH, _CH)
        return (pltpu.async_copy(x_hbm.at[wid, src], bufs[slot][0], sem),
                pltpu.async_copy(x_hbm.at[_B + wid, src], bufs[slot][1], sem),
                pltpu.async_copy(x_hbm.at[2 * _B + wid, src], bufs[slot][2], sem))

    bufs = ((bx0, by0, bz0), (bx1, by1, bz1))
    sems = (sem0, sem1)
    cur = start(0, 0)
    for c in range(_NCHUNK):
        nxt = None
        if c + 1 < _NCHUNK:
            nxt = start(c + 1, (c + 1) % 2)
        for h in cur:
            h.wait()
        process(*bufs[c % 2])
        cur = nxt

    for j in range(_V // _L):
        acc = zeros
        for l in range(_L):
            acc = acc + plsc.load_gather(hist, [iota * _L + (j * _L * _L + l)])
        counts[pl.ds(j * _L, _L)] = acc

    acc0 = zeros
    acc1 = zeros
    acc2 = zeros
    for v in range(_V):
        cv = plsc.load_gather(counts, [jnp.full((_L,), v, jnp.int32)])
        acc0 = acc0 + cv * wv[pl.ds(v * _CPAD, _L)]
        acc1 = acc1 + cv * wv[pl.ds(v * _CPAD + _L, _L)]
        acc2 = acc2 + cv * wv[pl.ds(v * _CPAD + 2 * _L, _L)]
    scale = jnp.float32(1.0 / _N)
    ov[pl.ds(0, _L)] = acc0 * scale + bv[pl.ds(0, _L)]
    ov[pl.ds(_L, _L)] = acc1 * scale + bv[pl.ds(_L, _L)]
    ov[pl.ds(2 * _L, _L)] = acc2 * scale + bv[pl.ds(2 * _L, _L)]
    pltpu.sync_copy(ov, out_hbm.at[wid])


@jax.jit
def _histogram_classify(xt, wp, bp):
    mesh = plsc.VectorSubcoreMesh(core_axis_name="c", subcore_axis_name="s")
    fn = functools.partial(
        pl.kernel,
        mesh=mesh,
        compiler_params=pltpu.CompilerParams(
            needs_layout_passes=False, use_tc_tiling_on_sc=True),
        out_type=jax.ShapeDtypeStruct((_B, _CPAD), jnp.float32),
        scratch_types=[
            pltpu.VMEM((_CH,), jnp.float32),
            pltpu.VMEM((_CH,), jnp.float32),
            pltpu.VMEM((_CH,), jnp.float32),
            pltpu.VMEM((_CH,), jnp.float32),
            pltpu.VMEM((_CH,), jnp.float32),
            pltpu.VMEM((_CH,), jnp.float32),
            pltpu.VMEM((_L * _V,), jnp.float32),
            pltpu.VMEM((_V,), jnp.float32),
            pltpu.VMEM((_V * _CPAD,), jnp.float32),
            pltpu.VMEM((_CPAD,), jnp.float32),
            pltpu.VMEM((_CPAD,), jnp.float32),
            pltpu.SemaphoreType.DMA,
            pltpu.SemaphoreType.DMA,
        ],
    )(_sc_body)
    return fn(xt, wp, bp)


def kernel(x, W, b):
    xt = jnp.transpose(x, (2, 0, 1)).reshape(3 * _B, _N)
    wp = jnp.zeros((_V, _CPAD), jnp.float32).at[:, :_C].set(W).reshape(-1)
    bp = jnp.zeros((_CPAD,), jnp.float32).at[:_C].set(b)
    out = _histogram_classify(xt, wp, bp)
    return out[:, :_C]


# speedup vs baseline: 167.3593x; 1.0248x over previous
import functools

import jax
import jax.numpy as jnp
from jax import lax
from jax.experimental import pallas as pl
from jax.experimental.pallas import tpu as pltpu
from jax.experimental.pallas import tpu_sc as plsc

_B = 32
_N = 65536
_RES = 4
_V = _RES ** 3
_C = 40
_CPAD = 48
_L = 16

_CH = 4096
_NCHUNK = _N // _CH
_GROUPS = _CH // _L


def _sc_body(x_hbm, w_hbm, b_hbm, out_hbm,
             bx0, by0, bz0, bx1, by1, bz1,
             hist, counts, wv, bv, ov, sem0, sem1):
    ncores = 2
    wid = lax.axis_index("s") * ncores + lax.axis_index("c")

    iota = lax.iota(jnp.int32, _L)
    ones = jnp.ones((_L,), jnp.float32)
    zeros = jnp.zeros((_L,), jnp.float32)

    def process(bx, by, bz):
        @plsc.parallel_loop(0, _GROUPS, unroll=8)
        def body(g):
            o = pl.ds(g * _L, _L)
            xg = bx[o]
            yg = by[o]
            zg = bz[o]
            dx = (xg * 4.0).astype(jnp.int32)
            dy = (yg * 4.0).astype(jnp.int32)
            dz = (zg * 4.0).astype(jnp.int32)
            sidx = dx * 256 + dy * 64 + dz * 16 + iota
            plsc.addupdate_scatter(hist, [sidx], ones)

    def start(c, slot):
        sem = sems[slot]
        src = pl.ds(c * _CH, _CH)
        return (pltpu.async_copy(x_hbm.at[wid, src], bufs[slot][0], sem),
                pltpu.async_copy(x_hbm.at[_B + wid, src], bufs[slot][1], sem),
                pltpu.async_copy(x_hbm.at[2 * _B + wid, src], bufs[slot][2], sem))

    bufs = ((bx0, by0, bz0), (bx1, by1, bz1))
    sems = (sem0, sem1)
    cur = start(0, 0)

    pltpu.sync_copy(w_hbm, wv)
    pltpu.sync_copy(b_hbm, bv)
    for j in range(_V):
        hist[pl.ds(j * _L, _L)] = zeros

    for c in range(_NCHUNK):
        nxt = None
        if c + 1 < _NCHUNK:
            nxt = start(c + 1, (c + 1) % 2)
        for h in cur:
            h.wait()
        process(*bufs[c % 2])
        cur = nxt

    for j in range(_V // _L):
        acc = zeros
        for l in range(_L):
            acc = acc + plsc.load_gather(hist, [iota * _L + (j * _L * _L + l)])
        counts[pl.ds(j * _L, _L)] = acc

    acc0 = zeros
    acc1 = zeros
    acc2 = zeros
    for v in range(_V):
        cv = plsc.load_gather(counts, [jnp.full((_L,), v, jnp.int32)])
        acc0 = acc0 + cv * wv[pl.ds(v * _CPAD, _L)]
        acc1 = acc1 + cv * wv[pl.ds(v * _CPAD + _L, _L)]
        acc2 = acc2 + cv * wv[pl.ds(v * _CPAD + 2 * _L, _L)]
    scale = jnp.float32(1.0 / _N)
    ov[pl.ds(0, _L)] = acc0 * scale + bv[pl.ds(0, _L)]
    ov[pl.ds(_L, _L)] = acc1 * scale + bv[pl.ds(_L, _L)]
    ov[pl.ds(2 * _L, _L)] = acc2 * scale + bv[pl.ds(2 * _L, _L)]
    pltpu.sync_copy(ov, out_hbm.at[wid])


@jax.jit
def _histogram_classify(xt, wp, bp):
    mesh = plsc.VectorSubcoreMesh(core_axis_name="c", subcore_axis_name="s")
    fn = functools.partial(
        pl.kernel,
        mesh=mesh,
        compiler_params=pltpu.CompilerParams(
            needs_layout_passes=False, use_tc_tiling_on_sc=True),
        out_type=jax.ShapeDtypeStruct((_B, _CPAD), jnp.float32),
        scratch_types=[
            pltpu.VMEM((_CH,), jnp.float32),
            pltpu.VMEM((_CH,), jnp.float32),
            pltpu.VMEM((_CH,), jnp.float32),
            pltpu.VMEM((_CH,), jnp.float32),
            pltpu.VMEM((_CH,), jnp.float32),
            pltpu.VMEM((_CH,), jnp.float32),
            pltpu.VMEM((_L * _V,), jnp.float32),
            pltpu.VMEM((_V,), jnp.float32),
            pltpu.VMEM((_V * _CPAD,), jnp.float32),
            pltpu.VMEM((_CPAD,), jnp.float32),
            pltpu.VMEM((_CPAD,), jnp.float32),
            pltpu.SemaphoreType.DMA,
            pltpu.SemaphoreType.DMA,
        ],
    )(_sc_body)
    return fn(xt, wp, bp)


def kernel(x, W, b):
    xt = jnp.transpose(x, (2, 0, 1)).reshape(3 * _B, _N)
    wp = jnp.pad(W, ((0, 0), (0, _CPAD - _C))).reshape(-1)
    bp = jnp.pad(b, (0, _CPAD - _C))
    return _histogram_classify(xt, wp, bp)[:, :_C]


# speedup vs baseline: 176.0733x; 1.0521x over previous
import functools

import jax
import jax.numpy as jnp
from jax import lax
from jax.experimental import pallas as pl
from jax.experimental.pallas import tpu as pltpu
from jax.experimental.pallas import tpu_sc as plsc

_B = 32
_N = 65536
_RES = 4
_V = _RES ** 3
_C = 40
_CPAD = 48
_L = 16

_CH = 8192
_NCHUNK = _N // _CH
_GROUPS = _CH // _L


def _sc_body(x_hbm, w_hbm, b_hbm, out_hbm,
             bx0, by0, bz0, bx1, by1, bz1,
             hist, counts, wv, bv, ov, sem0, sem1):
    ncores = 2
    wid = lax.axis_index("s") * ncores + lax.axis_index("c")

    iota = lax.iota(jnp.int32, _L)
    ones = jnp.ones((_L,), jnp.float32)
    zeros = jnp.zeros((_L,), jnp.float32)

    def process(bx, by, bz):
        @plsc.parallel_loop(0, _GROUPS, unroll=16)
        def body(g):
            o = pl.ds(g * _L, _L)
            xg = bx[o]
            yg = by[o]
            zg = bz[o]
            dx = (xg * 4.0).astype(jnp.int32)
            dy = (yg * 4.0).astype(jnp.int32)
            dz = (zg * 4.0).astype(jnp.int32)
            sidx = dx * 256 + dy * 64 + dz * 16 + iota
            plsc.addupdate_scatter(hist, [sidx], ones)

    def start(c, slot):
        sem = sems[slot]
        src = pl.ds(c * _CH, _CH)
        return (pltpu.async_copy(x_hbm.at[wid, src], bufs[slot][0], sem),
                pltpu.async_copy(x_hbm.at[_B + wid, src], bufs[slot][1], sem),
                pltpu.async_copy(x_hbm.at[2 * _B + wid, src], bufs[slot][2], sem))

    bufs = ((bx0, by0, bz0), (bx1, by1, bz1))
    sems = (sem0, sem1)
    cur = start(0, 0)

    pltpu.sync_copy(w_hbm, wv)
    pltpu.sync_copy(b_hbm, bv)
    for j in range(_V):
        hist[pl.ds(j * _L, _L)] = zeros

    for c in range(_NCHUNK):
        nxt = None
        if c + 1 < _NCHUNK:
            nxt = start(c + 1, (c + 1) % 2)
        for h in cur:
            h.wait()
        process(*bufs[c % 2])
        cur = nxt

    for j in range(_V // _L):
        acc = zeros
        for l in range(_L):
            acc = acc + plsc.load_gather(hist, [iota * _L + (j * _L * _L + l)])
        counts[pl.ds(j * _L, _L)] = acc

    acc0 = zeros
    acc1 = zeros
    acc2 = zeros
    for v in range(_V):
        cv = plsc.load_gather(counts, [jnp.full((_L,), v, jnp.int32)])
        acc0 = acc0 + cv * wv[pl.ds(v * _CPAD, _L)]
        acc1 = acc1 + cv * wv[pl.ds(v * _CPAD + _L, _L)]
        acc2 = acc2 + cv * wv[pl.ds(v * _CPAD + 2 * _L, _L)]
    scale = jnp.float32(1.0 / _N)
    ov[pl.ds(0, _L)] = acc0 * scale + bv[pl.ds(0, _L)]
    ov[pl.ds(_L, _L)] = acc1 * scale + bv[pl.ds(_L, _L)]
    ov[pl.ds(2 * _L, _L)] = acc2 * scale + bv[pl.ds(2 * _L, _L)]
    pltpu.sync_copy(ov, out_hbm.at[wid])


@jax.jit
def _histogram_classify(xt, wp, bp):
    mesh = plsc.VectorSubcoreMesh(core_axis_name="c", subcore_axis_name="s")
    fn = functools.partial(
        pl.kernel,
        mesh=mesh,
        compiler_params=pltpu.CompilerParams(
            needs_layout_passes=False, use_tc_tiling_on_sc=True),
        out_type=jax.ShapeDtypeStruct((_B, _CPAD), jnp.float32),
        scratch_types=[
            pltpu.VMEM((_CH,), jnp.float32),
            pltpu.VMEM((_CH,), jnp.float32),
            pltpu.VMEM((_CH,), jnp.float32),
            pltpu.VMEM((_CH,), jnp.float32),
            pltpu.VMEM((_CH,), jnp.float32),
            pltpu.VMEM((_CH,), jnp.float32),
            pltpu.VMEM((_L * _V,), jnp.float32),
            pltpu.VMEM((_V,), jnp.float32),
            pltpu.VMEM((_V * _CPAD,), jnp.float32),
            pltpu.VMEM((_CPAD,), jnp.float32),
            pltpu.VMEM((_CPAD,), jnp.float32),
            pltpu.SemaphoreType.DMA,
            pltpu.SemaphoreType.DMA,
        ],
    )(_sc_body)
    return fn(xt, wp, bp)


def kernel(x, W, b):
    xt = jnp.transpose(x, (2, 0, 1)).reshape(3 * _B, _N)
    wp = jnp.pad(W, ((0, 0), (0, _CPAD - _C))).reshape(-1)
    bp = jnp.pad(b, (0, _CPAD - _C))
    return _histogram_classify(xt, wp, bp)[:, :_C]
